# Initial kernel scaffold; baseline (speedup 1.0000x reference)
import functools

import jax
import jax.numpy as jnp
from jax import lax
from jax.experimental import pallas as pl
from jax.experimental.pallas import tpu as pltpu
from jax.experimental.pallas import tpu_sc as plsc

B = 262144
Q = 262144
TI = 32
N = 16
NB = 14
M = 2 ** NB

NCORE = 2
NSUB = 16
BH = B // 2
QH = Q // 2
CH = 8192

RB = 1024
SENT = jnp.float32(-2.0 ** 40)


def _prep_body(mapping_ref, ctx_ref, qry_ref, tgt_ref,
               addrT_ref, qaddrT_ref, tgtT_ref):
    mp = mapping_ref[...]
    i3 = lax.broadcasted_iota(jnp.int32, (TI, N, NB), 0)
    p3 = jnp.int32(1) << lax.broadcasted_iota(jnp.int32, (TI, N, NB), 2)
    w = jnp.sum(jnp.where(mp[None] == i3, p3, 0), axis=2)
    wf = w.astype(jnp.float32)

    ctx = ctx_ref[...].astype(jnp.float32)
    addr = jax.lax.dot(ctx, wf, precision=lax.Precision.HIGHEST,
                       preferred_element_type=jnp.float32)
    addrT_ref[...] = addr.T.astype(jnp.int32)

    qry = qry_ref[...].astype(jnp.float32)
    qaddr = jax.lax.dot(qry, wf, precision=lax.Precision.HIGHEST,
                        preferred_element_type=jnp.float32)
    qaddrT_ref[...] = qaddr.T.astype(jnp.int32)

    tgtT_ref[...] = tgt_ref[...].T


def _prep(ctx_bits, query_bits, target_bits, mapping):
    grid = (B // RB,)
    return pl.pallas_call(
        _prep_body,
        grid=grid,
        in_specs=[
            pl.BlockSpec((N, NB), lambda i: (0, 0)),
            pl.BlockSpec((RB, TI), lambda i: (i, 0)),
            pl.BlockSpec((RB, TI), lambda i: (i, 0)),
            pl.BlockSpec((RB, N), lambda i: (i, 0)),
        ],
        out_specs=[
            pl.BlockSpec((N, RB), lambda i: (0, i)),
            pl.BlockSpec((N, RB), lambda i: (0, i)),
            pl.BlockSpec((N, RB), lambda i: (0, i)),
        ],
        out_shape=[
            jax.ShapeDtypeStruct((N, B), jnp.int32),
            jax.ShapeDtypeStruct((N, Q), jnp.int32),
            jax.ShapeDtypeStruct((N, B), jnp.float32),
        ],
    )(mapping, ctx_bits, query_bits, target_bits)


def _sc_body(mem_hbm, addrT_hbm, tgtT_hbm, qaddrT_hbm, outT_hbm,
             table, table2, abuf, tbuf, qbuf, obuf, shared):
    c = lax.axis_index("c")
    s = lax.axis_index("s")
    nn_loc = s % (N // NCORE)
    half = s // (N // NCORE)
    nn = c * (N // NCORE) + nn_loc

    @pl.when(half == 0)
    def _():
        pltpu.sync_copy(mem_hbm.at[nn], table)

    @pl.when(half == 1)
    def _():
        def fill(i, carry):
            table[pl.ds(i * 16, 16)] = jnp.full((16,), SENT, jnp.float32)
            return carry
        lax.fori_loop(0, M // 16, fill, 0)

    cbase = half * BH

    def commit_chunk(g, carry):
        off = cbase + g * CH
        pltpu.sync_copy(addrT_hbm.at[nn, pl.ds(off, CH)], abuf)
        pltpu.sync_copy(tgtT_hbm.at[nn, pl.ds(off, CH)], tbuf)

        def body(i, c2):
            a = abuf[pl.ds(i * 16, 16)]
            v = tbuf[pl.ds(i * 16, 16)]
            _, last = plsc.scan_count(a)
            plsc.store_scatter(table, [a], v, mask=last)
            return c2
        lax.fori_loop(0, CH // 16, body, 0)
        return carry
    lax.fori_loop(0, BH // CH, commit_chunk, 0)

    plsc.subcore_barrier()
    pltpu.sync_copy(table, shared.at[s])
    plsc.subcore_barrier()
    sib = jnp.where(half == 0, s + N // NCORE, s - N // NCORE)
    pltpu.sync_copy(shared.at[sib], table2)

    def merge(i, carry):
        sl = pl.ds(i * 16, 16)
        mine = table[sl]
        other = table2[sl]
        hi = jnp.where(half == 1, mine, other)
        lo = jnp.where(half == 1, other, mine)
        table[sl] = jnp.where(hi != SENT, hi, lo)
        return carry
    lax.fori_loop(0, M // 16, merge, 0)

    qbase = half * QH

    def query_chunk(g, carry):
        off = qbase + g * CH
        pltpu.sync_copy(qaddrT_hbm.at[nn, pl.ds(off, CH)], qbuf)

        def body(i, c2):
            qa = qbuf[pl.ds(i * 16, 16)]
            obuf[pl.ds(i * 16, 16)] = plsc.load_gather(table, [qa])
            return c2
        lax.fori_loop(0, CH // 16, body, 0)
        pltpu.sync_copy(obuf, outT_hbm.at[nn, pl.ds(off, CH)])
        return carry
    lax.fori_loop(0, QH // CH, query_chunk, 0)


def _sc_run(memory, addrT, tgtT, qaddrT):
    mesh = plsc.VectorSubcoreMesh(core_axis_name="c", subcore_axis_name="s")
    run = pl.kernel(
        _sc_body,
        out_type=jax.ShapeDtypeStruct((N, Q), jnp.float32),
        mesh=mesh,
        scratch_types=[
            pltpu.VMEM((M,), jnp.float32),
            pltpu.VMEM((M,), jnp.float32),
            pltpu.VMEM((CH,), jnp.int32),
            pltpu.VMEM((CH,), jnp.float32),
            pltpu.VMEM((CH,), jnp.int32),
            pltpu.VMEM((CH,), jnp.float32),
            pltpu.VMEM_SHARED((NSUB, M), jnp.float32),
        ],
    )
    return run(memory, addrT, tgtT, qaddrT)


def _post_body(outT_ref, out_ref):
    out_ref[...] = outT_ref[...].T


def _post(outT):
    return pl.pallas_call(
        _post_body,
        grid=(Q // RB,),
        in_specs=[pl.BlockSpec((N, RB), lambda i: (0, i))],
        out_specs=pl.BlockSpec((RB, N), lambda i: (i, 0)),
        out_shape=jax.ShapeDtypeStruct((Q, N), jnp.float32),
    )(outT)


def kernel(memory, target_bits, ctx_bits, query_bits, mapping):
    addrT, qaddrT, tgtT = _prep(ctx_bits, query_bits, target_bits, mapping)
    outT = _sc_run(memory, addrT, tgtT, qaddrT)
    return _post(outT)


# speedup vs baseline: 60.4008x; 60.4008x over previous
import functools

import jax
import jax.numpy as jnp
from jax import lax
from jax.experimental import pallas as pl
from jax.experimental.pallas import tpu as pltpu
from jax.experimental.pallas import tpu_sc as plsc

B = 262144
Q = 262144
TI = 32
N = 16
NB = 14
M = 2 ** NB

NCORE = 2
NSUB = 16
BH = B // 2
QH = Q // 2
CH = 8192

RB = 1024
SENT = -(2.0 ** 40)


def _prep_body(mapping_ref, ctx_ref, qry_ref, tgt_ref,
               addrT_ref, qaddrT_ref, tgtT_ref):
    mp = mapping_ref[...]
    i3 = lax.broadcasted_iota(jnp.int32, (TI, N, NB), 0)
    p3 = jnp.int32(1) << lax.broadcasted_iota(jnp.int32, (TI, N, NB), 2)
    w = jnp.sum(jnp.where(mp[None] == i3, p3, 0), axis=2)
    wf = w.astype(jnp.float32)

    ctx = ctx_ref[...].astype(jnp.float32)
    addr = jax.lax.dot(ctx, wf, precision=lax.Precision.HIGHEST,
                       preferred_element_type=jnp.float32)
    addrT_ref[...] = addr.T.astype(jnp.int32)

    qry = qry_ref[...].astype(jnp.float32)
    qaddr = jax.lax.dot(qry, wf, precision=lax.Precision.HIGHEST,
                        preferred_element_type=jnp.float32)
    qaddrT_ref[...] = qaddr.T.astype(jnp.int32)

    tgtT_ref[...] = tgt_ref[...].T


def _prep(ctx_bits, query_bits, target_bits, mapping):
    grid = (B // RB,)
    return pl.pallas_call(
        _prep_body,
        grid=grid,
        in_specs=[
            pl.BlockSpec((N, NB), lambda i: (0, 0)),
            pl.BlockSpec((RB, TI), lambda i: (i, 0)),
            pl.BlockSpec((RB, TI), lambda i: (i, 0)),
            pl.BlockSpec((RB, N), lambda i: (i, 0)),
        ],
        out_specs=[
            pl.BlockSpec((N, RB), lambda i: (0, i)),
            pl.BlockSpec((N, RB), lambda i: (0, i)),
            pl.BlockSpec((N, RB), lambda i: (0, i)),
        ],
        out_shape=[
            jax.ShapeDtypeStruct((N, B), jnp.int32),
            jax.ShapeDtypeStruct((N, Q), jnp.int32),
            jax.ShapeDtypeStruct((N, B), jnp.float32),
        ],
    )(mapping, ctx_bits, query_bits, target_bits)


def _sc_body(mem_hbm, addrT_hbm, tgtT_hbm, qaddrT_hbm, outT_hbm,
             table, table2, abuf, tbuf, qbuf, obuf, shared):
    c = lax.axis_index("c")
    s = lax.axis_index("s")
    nn_loc = s % (N // NCORE)
    half = s // (N // NCORE)
    nn = c * (N // NCORE) + nn_loc

    @pl.when(half == 0)
    def _():
        pltpu.sync_copy(mem_hbm.at[nn], table)

    @pl.when(half == 1)
    def _():
        def fill(i, carry):
            table[pl.ds(i * 16, 16)] = jnp.full((16,), SENT, jnp.float32)
            return carry
        lax.fori_loop(0, M // 16, fill, 0)

    cbase = half * BH

    def commit_chunk(g, carry):
        off = cbase + g * CH
        pltpu.sync_copy(addrT_hbm.at[nn, pl.ds(off, CH)], abuf)
        pltpu.sync_copy(tgtT_hbm.at[nn, pl.ds(off, CH)], tbuf)

        def body(i, c2):
            a = abuf[pl.ds(i * 16, 16)]
            v = tbuf[pl.ds(i * 16, 16)]
            _, last = plsc.scan_count(a)
            plsc.store_scatter(table, [a], v, mask=last)
            return c2
        lax.fori_loop(0, CH // 16, body, 0)
        return carry
    lax.fori_loop(0, BH // CH, commit_chunk, 0)

    plsc.subcore_barrier()
    pltpu.sync_copy(table, shared.at[s])
    plsc.subcore_barrier()
    sib = jnp.where(half == 0, s + N // NCORE, s - N // NCORE)
    pltpu.sync_copy(shared.at[sib], table2)

    def merge(i, carry):
        sl = pl.ds(i * 16, 16)
        mine = table[sl]
        other = table2[sl]
        hi = jnp.where(half == 1, mine, other)
        lo = jnp.where(half == 1, other, mine)
        table[sl] = jnp.where(hi != SENT, hi, lo)
        return carry
    lax.fori_loop(0, M // 16, merge, 0)

    qbase = half * QH

    def query_chunk(g, carry):
        off = qbase + g * CH
        pltpu.sync_copy(qaddrT_hbm.at[nn, pl.ds(off, CH)], qbuf)

        def body(i, c2):
            qa = qbuf[pl.ds(i * 16, 16)]
            obuf[pl.ds(i * 16, 16)] = plsc.load_gather(table, [qa])
            return c2
        lax.fori_loop(0, CH // 16, body, 0)
        pltpu.sync_copy(obuf, outT_hbm.at[nn, pl.ds(off, CH)])
        return carry
    lax.fori_loop(0, QH // CH, query_chunk, 0)


def _sc_run(memory, addrT, tgtT, qaddrT):
    mesh = plsc.VectorSubcoreMesh(core_axis_name="c", subcore_axis_name="s")
    run = pl.kernel(
        _sc_body,
        out_type=jax.ShapeDtypeStruct((N, Q), jnp.float32),
        mesh=mesh,
        scratch_types=[
            pltpu.VMEM((M,), jnp.float32),
            pltpu.VMEM((M,), jnp.float32),
            pltpu.VMEM((CH,), jnp.int32),
            pltpu.VMEM((CH,), jnp.float32),
            pltpu.VMEM((CH,), jnp.int32),
            pltpu.VMEM((CH,), jnp.float32),
            pltpu.VMEM_SHARED((NSUB, M), jnp.float32),
        ],
        compiler_params=pltpu.CompilerParams(needs_layout_passes=False),
    )
    return run(memory, addrT, tgtT, qaddrT)


def _post_body(outT_ref, out_ref):
    out_ref[...] = outT_ref[...].T


def _post(outT):
    return pl.pallas_call(
        _post_body,
        grid=(Q // RB,),
        in_specs=[pl.BlockSpec((N, RB), lambda i: (0, i))],
        out_specs=pl.BlockSpec((RB, N), lambda i: (i, 0)),
        out_shape=jax.ShapeDtypeStruct((Q, N), jnp.float32),
    )(outT)


def kernel(memory, target_bits, ctx_bits, query_bits, mapping):
    addrT, qaddrT, tgtT = _prep(ctx_bits, query_bits, target_bits, mapping)
    outT = _sc_run(memory, addrT, tgtT, qaddrT)
    return _post(outT)


# speedup vs baseline: 83.0295x; 1.3746x over previous
import functools

import jax
import jax.numpy as jnp
from jax import lax
from jax.experimental import pallas as pl
from jax.experimental.pallas import tpu as pltpu
from jax.experimental.pallas import tpu_sc as plsc

B = 262144
Q = 262144
TI = 32
N = 16
NB = 14
M = 2 ** NB

NCORE = 2
NSUB = 16
BH = B // 2
QH = Q // 2
CH = 8192

RB = 4096
SENT = -(2.0 ** 40)


def _prep_body(mapping_ref, ctx_ref, qry_ref, tgt_ref,
               addrT_ref, qaddrT_ref, tgtT_ref):
    mp = mapping_ref[...]
    i3 = lax.broadcasted_iota(jnp.int32, (TI, 2 * N, NB), 0)
    n3 = lax.broadcasted_iota(jnp.int32, (TI, 2 * N, NB), 1)
    k3 = lax.broadcasted_iota(jnp.int32, (TI, 2 * N, NB), 2)
    sel_lo = (n3 < N) & (k3 < 7)
    valid = sel_lo | ((n3 >= N) & (k3 >= 7))
    shift = jnp.where(valid, jnp.where(sel_lo, k3, k3 - 7), 0)
    p3 = jnp.where(valid, jnp.int32(1) << shift, 0)
    mp2 = jnp.concatenate([mp, mp], axis=0)
    w2 = jnp.sum(jnp.where(mp2[None] == i3, p3, 0), axis=2)
    wb = w2.astype(jnp.bfloat16)

    def addresses(bits_i32):
        bits = bits_i32.astype(jnp.bfloat16)
        r = jax.lax.dot(bits, wb,
                        preferred_element_type=jnp.float32)
        a = r[:, :N] + 128.0 * r[:, N:]
        return a.astype(jnp.int32)

    addrT_ref[...] = addresses(ctx_ref[...]).T
    qaddrT_ref[...] = addresses(qry_ref[...]).T
    tgtT_ref[...] = tgt_ref[...].T


def _prep(ctx_bits, query_bits, target_bits, mapping):
    grid = (B // RB,)
    return pl.pallas_call(
        _prep_body,
        grid=grid,
        in_specs=[
            pl.BlockSpec((N, NB), lambda i: (0, 0)),
            pl.BlockSpec((RB, TI), lambda i: (i, 0)),
            pl.BlockSpec((RB, TI), lambda i: (i, 0)),
            pl.BlockSpec((RB, N), lambda i: (i, 0)),
        ],
        out_specs=[
            pl.BlockSpec((N, RB), lambda i: (0, i)),
            pl.BlockSpec((N, RB), lambda i: (0, i)),
            pl.BlockSpec((N, RB), lambda i: (0, i)),
        ],
        out_shape=[
            jax.ShapeDtypeStruct((N, B), jnp.int32),
            jax.ShapeDtypeStruct((N, Q), jnp.int32),
            jax.ShapeDtypeStruct((N, B), jnp.float32),
        ],
    )(mapping, ctx_bits, query_bits, target_bits)


def _sc_body(mem_hbm, addrT_hbm, tgtT_hbm, qaddrT_hbm, outT_hbm,
             table, table2, abuf, tbuf, qbuf, obuf, shared,
             semc0, semc1, semq0, semq1, semo0, semo1):
    c = lax.axis_index("c")
    s = lax.axis_index("s")
    nn_loc = s % (N // NCORE)
    half = s // (N // NCORE)
    nn = c * (N // NCORE) + nn_loc

    semc = [semc0, semc1]
    semq = [semq0, semq1]
    semo = [semo0, semo1]

    cbase = half * BH
    GC = BH // CH

    def commit_start(g):
        slot = g % 2
        off = cbase + g * CH
        ha = pltpu.async_copy(addrT_hbm.at[nn, pl.ds(off, CH)],
                              abuf.at[slot], semc[slot])
        hv = pltpu.async_copy(tgtT_hbm.at[nn, pl.ds(off, CH)],
                              tbuf.at[slot], semc[slot])
        return (ha, hv)

    pend_c = [commit_start(0), commit_start(1)]

    @pl.when(half == 0)
    def _():
        pltpu.sync_copy(mem_hbm.at[nn], table)

    @pl.when(half == 1)
    def _():
        def fill(i, carry):
            table[pl.ds(i * 16, 16)] = jnp.full((16,), SENT, jnp.float32)
            return carry
        lax.fori_loop(0, M // 16, fill, 0)

    for g in range(GC):
        slot = g % 2
        ha, hv = pend_c[g]
        ha.wait()
        hv.wait()

        def commit_body(i, c2, _slot=slot):
            base = i * 64
            for u in range(4):
                sl = pl.ds(base + u * 16, 16)
                a = abuf[_slot, sl]
                v = tbuf[_slot, sl]
                _, last = plsc.scan_count(a)
                plsc.store_scatter(table, [a], v, mask=last)
            return c2
        lax.fori_loop(0, CH // 64, commit_body, 0)
        if g + 2 < GC:
            pend_c.append(commit_start(g + 2))

    qbase = half * QH
    GQ = QH // CH

    def query_start(g):
        slot = g % 2
        off = qbase + g * CH
        return pltpu.async_copy(qaddrT_hbm.at[nn, pl.ds(off, CH)],
                                qbuf.at[slot], semq[slot])

    pend_q = [query_start(0), query_start(1)]

    plsc.subcore_barrier()
    pltpu.sync_copy(table, shared.at[s])
    plsc.subcore_barrier()
    sib = jnp.where(half == 0, s + N // NCORE, s - N // NCORE)
    pltpu.sync_copy(shared.at[sib], table2)

    def merge(i, carry):
        base = i * 64
        for u in range(4):
            sl = pl.ds(base + u * 16, 16)
            mine = table[sl]
            other = table2[sl]
            hi = jnp.where(half == 1, mine, other)
            lo = jnp.where(half == 1, other, mine)
            table[sl] = jnp.where(hi != SENT, hi, lo)
        return carry
    lax.fori_loop(0, M // 64, merge, 0)

    pend_o = [None, None]
    for g in range(GQ):
        slot = g % 2
        pend_q[g].wait()
        if pend_o[slot] is not None:
            pend_o[slot].wait()

        def query_body(i, c2, _slot=slot):
            base = i * 64
            for u in range(4):
                sl = pl.ds(base + u * 16, 16)
                qa = qbuf[_slot, sl]
                obuf[_slot, sl] = plsc.load_gather(table, [qa])
            return c2
        lax.fori_loop(0, CH // 64, query_body, 0)

        off = qbase + g * CH
        pend_o[slot] = pltpu.async_copy(
            obuf.at[slot], outT_hbm.at[nn, pl.ds(off, CH)], semo[slot])
        if g + 2 < GQ:
            pend_q.append(query_start(g + 2))
    pend_o[0].wait()
    pend_o[1].wait()


def _sc_run(memory, addrT, tgtT, qaddrT):
    mesh = plsc.VectorSubcoreMesh(core_axis_name="c", subcore_axis_name="s")
    run = pl.kernel(
        _sc_body,
        out_type=jax.ShapeDtypeStruct((N, Q), jnp.float32),
        mesh=mesh,
        scratch_types=[
            pltpu.VMEM((M,), jnp.float32),
            pltpu.VMEM((M,), jnp.float32),
            pltpu.VMEM((2, CH), jnp.int32),
            pltpu.VMEM((2, CH), jnp.float32),
            pltpu.VMEM((2, CH), jnp.int32),
            pltpu.VMEM((2, CH), jnp.float32),
            pltpu.VMEM_SHARED((NSUB, M), jnp.float32),
            pltpu.SemaphoreType.DMA,
            pltpu.SemaphoreType.DMA,
            pltpu.SemaphoreType.DMA,
            pltpu.SemaphoreType.DMA,
            pltpu.SemaphoreType.DMA,
            pltpu.SemaphoreType.DMA,
        ],
        compiler_params=pltpu.CompilerParams(needs_layout_passes=False),
    )
    return run(memory, addrT, tgtT, qaddrT)


def _post_body(outT_ref, out_ref):
    out_ref[...] = outT_ref[...].T


def _post(outT):
    return pl.pallas_call(
        _post_body,
        grid=(Q // RB,),
        in_specs=[pl.BlockSpec((N, RB), lambda i: (0, i))],
        out_specs=pl.BlockSpec((RB, N), lambda i: (i, 0)),
        out_shape=jax.ShapeDtypeStruct((Q, N), jnp.float32),
    )(outT)


def kernel(memory, target_bits, ctx_bits, query_bits, mapping):
    addrT, qaddrT, tgtT = _prep(ctx_bits, query_bits, target_bits, mapping)
    outT = _sc_run(memory, addrT, tgtT, qaddrT)
    return _post(outT)


# speedup vs baseline: 86.5722x; 1.0427x over previous
import jax
import jax.numpy as jnp
from jax import lax
from jax.experimental import pallas as pl
from jax.experimental.pallas import tpu as pltpu
from jax.experimental.pallas import tpu_sc as plsc

B = 262144
Q = 262144
TI = 32
N = 16
NB = 14
M = 2 ** NB

NCORE = 2
NSUB = 16
BH = B // 2
QH = Q // 2
CH = 4096
G = B // CH
SENT = -(2.0 ** 40)


def _prep_body(mapping_ref, ctx_ref, qry_ref, tgt_ref,
               addrT_ref, qaddrT_ref, tgtT_ref):
    mp = mapping_ref[...]
    i3 = lax.broadcasted_iota(jnp.int32, (TI, 2 * N, NB), 0)
    n3 = lax.broadcasted_iota(jnp.int32, (TI, 2 * N, NB), 1)
    k3 = lax.broadcasted_iota(jnp.int32, (TI, 2 * N, NB), 2)
    sel_lo = (n3 < N) & (k3 < 7)
    valid = sel_lo | ((n3 >= N) & (k3 >= 7))
    shift = jnp.where(valid, jnp.where(sel_lo, k3, k3 - 7), 0)
    p3 = jnp.where(valid, jnp.int32(1) << shift, 0)
    mp2 = jnp.concatenate([mp, mp], axis=0)
    w2 = jnp.sum(jnp.where(mp2[None] == i3, p3, 0), axis=2)
    wb = w2.astype(jnp.bfloat16)

    def addresses(bits_i32):
        bits = bits_i32.astype(jnp.bfloat16)
        r = jax.lax.dot(bits, wb,
                        preferred_element_type=jnp.float32)
        rt = r.T
        a = rt[:N] + 128.0 * rt[N:]
        return a.astype(jnp.int32)

    addrT_ref[...] = addresses(ctx_ref[...])[None]
    qaddrT_ref[...] = addresses(qry_ref[...])[None]
    tgtT_ref[...] = tgt_ref[...].T[None]


def _prep(ctx_bits, query_bits, target_bits, mapping):
    return pl.pallas_call(
        _prep_body,
        grid=(G,),
        in_specs=[
            pl.BlockSpec((N, NB), lambda i: (0, 0)),
            pl.BlockSpec((CH, TI), lambda i: (i, 0)),
            pl.BlockSpec((CH, TI), lambda i: (i, 0)),
            pl.BlockSpec((CH, N), lambda i: (i, 0)),
        ],
        out_specs=[
            pl.BlockSpec((1, N, CH), lambda i: (i, 0, 0)),
            pl.BlockSpec((1, N, CH), lambda i: (i, 0, 0)),
            pl.BlockSpec((1, N, CH), lambda i: (i, 0, 0)),
        ],
        out_shape=[
            jax.ShapeDtypeStruct((G, N, CH), jnp.int32),
            jax.ShapeDtypeStruct((G, N, CH), jnp.int32),
            jax.ShapeDtypeStruct((G, N, CH), jnp.float32),
        ],
    )(mapping, ctx_bits, query_bits, target_bits)


def _sc_body(mem_hbm, addrT_hbm, tgtT_hbm, qaddrT_hbm, outT_hbm,
             table, table2, abuf, tbuf, qbuf, obuf, shared,
             semc0, semc1, semq0, semq1, semo0, semo1):
    c = lax.axis_index("c")
    s = lax.axis_index("s")
    nn_loc = s % (N // NCORE)
    half = s // (N // NCORE)
    nn = c * (N // NCORE) + nn_loc

    semc = [semc0, semc1]
    semq = [semq0, semq1]
    semo = [semo0, semo1]

    gbase = half * (G // 2)
    GC = G // 2

    def commit_start(g):
        slot = g % 2
        ha = pltpu.async_copy(addrT_hbm.at[gbase + g, nn],
                              abuf.at[slot], semc[slot])
        hv = pltpu.async_copy(tgtT_hbm.at[gbase + g, nn],
                              tbuf.at[slot], semc[slot])
        return (ha, hv)

    pend_c = [commit_start(0), commit_start(1)]

    @pl.when(half == 0)
    def _():
        pltpu.sync_copy(mem_hbm.at[nn], table)

    @pl.when(half == 1)
    def _():
        def fill(i, carry):
            table[pl.ds(i * 16, 16)] = jnp.full((16,), SENT, jnp.float32)
            return carry
        lax.fori_loop(0, M // 16, fill, 0)

    for g in range(GC):
        slot = g % 2
        ha, hv = pend_c[g]
        ha.wait()
        hv.wait()

        def commit_body(i, c2, _slot=slot):
            base = i * 64
            for u in range(4):
                sl = pl.ds(base + u * 16, 16)
                a = abuf[_slot, sl]
                v = tbuf[_slot, sl]
                _, last = plsc.scan_count(a)
                plsc.store_scatter(table, [a], v, mask=last)
            return c2
        lax.fori_loop(0, CH // 64, commit_body, 0)
        if g + 2 < GC:
            pend_c.append(commit_start(g + 2))

    GQ = G // 2

    def query_start(g):
        slot = g % 2
        return pltpu.async_copy(qaddrT_hbm.at[gbase + g, nn],
                                qbuf.at[slot], semq[slot])

    pend_q = [query_start(0), query_start(1)]

    plsc.subcore_barrier()
    pltpu.sync_copy(table, shared.at[s])
    plsc.subcore_barrier()
    sib = jnp.where(half == 0, s + N // NCORE, s - N // NCORE)
    pltpu.sync_copy(shared.at[sib], table2)

    def merge(i, carry):
        base = i * 64
        for u in range(4):
            sl = pl.ds(base + u * 16, 16)
            mine = table[sl]
            other = table2[sl]
            hi = jnp.where(half == 1, mine, other)
            lo = jnp.where(half == 1, other, mine)
            table[sl] = jnp.where(hi != SENT, hi, lo)
        return carry
    lax.fori_loop(0, M // 64, merge, 0)

    pend_o = [None, None]
    for g in range(GQ):
        slot = g % 2
        pend_q[g].wait()
        if pend_o[slot] is not None:
            pend_o[slot].wait()

        def query_body(i, c2, _slot=slot):
            base = i * 64
            for u in range(4):
                sl = pl.ds(base + u * 16, 16)
                qa = qbuf[_slot, sl]
                obuf[_slot, sl] = plsc.load_gather(table, [qa])
            return c2
        lax.fori_loop(0, CH // 64, query_body, 0)

        pend_o[slot] = pltpu.async_copy(
            obuf.at[slot], outT_hbm.at[gbase + g, nn], semo[slot])
        if g + 2 < GQ:
            pend_q.append(query_start(g + 2))
    pend_o[0].wait()
    pend_o[1].wait()


def _sc_run(memory, addrT, tgtT, qaddrT):
    mesh = plsc.VectorSubcoreMesh(core_axis_name="c", subcore_axis_name="s")
    run = pl.kernel(
        _sc_body,
        out_type=jax.ShapeDtypeStruct((G, N, CH), jnp.float32),
        mesh=mesh,
        scratch_types=[
            pltpu.VMEM((M,), jnp.float32),
            pltpu.VMEM((M,), jnp.float32),
            pltpu.VMEM((2, CH), jnp.int32),
            pltpu.VMEM((2, CH), jnp.float32),
            pltpu.VMEM((2, CH), jnp.int32),
            pltpu.VMEM((2, CH), jnp.float32),
            pltpu.VMEM_SHARED((NSUB, M), jnp.float32),
            pltpu.SemaphoreType.DMA,
            pltpu.SemaphoreType.DMA,
            pltpu.SemaphoreType.DMA,
            pltpu.SemaphoreType.DMA,
            pltpu.SemaphoreType.DMA,
            pltpu.SemaphoreType.DMA,
        ],
        compiler_params=pltpu.CompilerParams(needs_layout_passes=False),
    )
    return run(memory, addrT, tgtT, qaddrT)


def _post_body(outT_ref, out_ref):
    out_ref[...] = outT_ref[0].T


def _post(outT):
    return pl.pallas_call(
        _post_body,
        grid=(G,),
        in_specs=[pl.BlockSpec((1, N, CH), lambda i: (i, 0, 0))],
        out_specs=pl.BlockSpec((CH, N), lambda i: (i, 0)),
        out_shape=jax.ShapeDtypeStruct((Q, N), jnp.float32),
    )(outT)


def kernel(memory, target_bits, ctx_bits, query_bits, mapping):
    addrT, qaddrT, tgtT = _prep(ctx_bits, query_bits, target_bits, mapping)
    outT = _sc_run(memory, addrT, tgtT, qaddrT)
    return _post(outT)


# speedup vs baseline: 86.9430x; 1.0043x over previous
import jax
import jax.numpy as jnp
from jax import lax
from jax.experimental import pallas as pl
from jax.experimental.pallas import tpu as pltpu
from jax.experimental.pallas import tpu_sc as plsc

B = 262144
Q = 262144
TI = 32
N = 16
NB = 14
M = 2 ** NB

NCORE = 2
NSUB = 16
BH = B // 2
QH = Q // 2
CH = 4096
G = B // CH
SENT = -(2.0 ** 40)


def _prep_body(mapping_ref, ctx_ref, qry_ref, tgt_ref,
               addrT_ref, qaddrT_ref, tgtT_ref):
    mp = mapping_ref[...]
    i3 = lax.broadcasted_iota(jnp.int32, (TI, 2 * N, NB), 0)
    n3 = lax.broadcasted_iota(jnp.int32, (TI, 2 * N, NB), 1)
    k3 = lax.broadcasted_iota(jnp.int32, (TI, 2 * N, NB), 2)
    sel_lo = (n3 < N) & (k3 < 7)
    valid = sel_lo | ((n3 >= N) & (k3 >= 7))
    shift = jnp.where(valid, jnp.where(sel_lo, k3, k3 - 7), 0)
    p3 = jnp.where(valid, jnp.int32(1) << shift, 0)
    mp2 = jnp.concatenate([mp, mp], axis=0)
    w2 = jnp.sum(jnp.where(mp2[None] == i3, p3, 0), axis=2)
    wb = w2.astype(jnp.bfloat16)

    def addresses(bits_i32):
        bits = bits_i32.astype(jnp.bfloat16)
        r = jax.lax.dot(bits, wb,
                        preferred_element_type=jnp.float32)
        rt = r.T
        a = rt[:N] + 128.0 * rt[N:]
        ai = a.astype(jnp.int32)
        return ai[:, :CH // 2] | (ai[:, CH // 2:] << 16)

    addrT_ref[...] = addresses(ctx_ref[...])[None]
    qaddrT_ref[...] = addresses(qry_ref[...])[None]
    tgtT_ref[...] = tgt_ref[...].T[None]


def _prep(ctx_bits, query_bits, target_bits, mapping):
    return pl.pallas_call(
        _prep_body,
        grid=(G,),
        in_specs=[
            pl.BlockSpec((N, NB), lambda i: (0, 0)),
            pl.BlockSpec((CH, TI), lambda i: (i, 0)),
            pl.BlockSpec((CH, TI), lambda i: (i, 0)),
            pl.BlockSpec((CH, N), lambda i: (i, 0)),
        ],
        out_specs=[
            pl.BlockSpec((1, N, CH // 2), lambda i: (i, 0, 0)),
            pl.BlockSpec((1, N, CH // 2), lambda i: (i, 0, 0)),
            pl.BlockSpec((1, N, CH), lambda i: (i, 0, 0)),
        ],
        out_shape=[
            jax.ShapeDtypeStruct((G, N, CH // 2), jnp.int32),
            jax.ShapeDtypeStruct((G, N, CH // 2), jnp.int32),
            jax.ShapeDtypeStruct((G, N, CH), jnp.float32),
        ],
    )(mapping, ctx_bits, query_bits, target_bits)


def _sc_body(mem_hbm, addrT_hbm, tgtT_hbm, qaddrT_hbm, outT_hbm,
             table, table2, abuf, tbuf, qbuf, obuf, shared,
             semc0, semc1, semq0, semq1, semo0, semo1):
    c = lax.axis_index("c")
    s = lax.axis_index("s")
    nn_loc = s % (N // NCORE)
    half = s // (N // NCORE)
    nn = c * (N // NCORE) + nn_loc

    semc = [semc0, semc1]
    semq = [semq0, semq1]
    semo = [semo0, semo1]

    gbase = half * (G // 2)
    GC = G // 2

    def commit_start(g):
        slot = g % 2
        ha = pltpu.async_copy(addrT_hbm.at[gbase + g, nn],
                              abuf.at[slot], semc[slot])
        hv = pltpu.async_copy(tgtT_hbm.at[gbase + g, nn],
                              tbuf.at[slot], semc[slot])
        return (ha, hv)

    pend_c = [commit_start(0), commit_start(1)]

    @pl.when(half == 0)
    def _():
        pltpu.sync_copy(mem_hbm.at[nn], table)

    @pl.when(half == 1)
    def _():
        def fill(i, carry):
            table[pl.ds(i * 16, 16)] = jnp.full((16,), SENT, jnp.float32)
            return carry
        lax.fori_loop(0, M // 16, fill, 0)

    for g in range(GC):
        slot = g % 2
        ha, hv = pend_c[g]
        ha.wait()
        hv.wait()

        def commit_body(i, c2, _slot=slot, _half=0):
            base = i * 64
            for u in range(4):
                sl = pl.ds(base + u * 16, 16)
                w = abuf[_slot, sl]
                a = lax.shift_right_logical(w, 16) if _half else (w & 0xFFFF)
                v = tbuf[_slot, pl.ds(_half * (CH // 2) + base + u * 16, 16)]
                _, last = plsc.scan_count(a)
                plsc.store_scatter(table, [a], v, mask=last)
            return c2
        lax.fori_loop(0, CH // 128, commit_body, 0)
        lax.fori_loop(0, CH // 128,
                      lambda i, c2, _s=slot: commit_body(i, c2, _s, 1), 0)
        if g + 2 < GC:
            pend_c.append(commit_start(g + 2))

    GQ = G // 2

    def query_start(g):
        slot = g % 2
        return pltpu.async_copy(qaddrT_hbm.at[gbase + g, nn],
                                qbuf.at[slot], semq[slot])

    pend_q = [query_start(0), query_start(1)]

    plsc.subcore_barrier()
    pltpu.sync_copy(table, shared.at[s])
    plsc.subcore_barrier()
    sib = jnp.where(half == 0, s + N // NCORE, s - N // NCORE)
    pltpu.sync_copy(shared.at[sib], table2)

    def merge(i, carry):
        base = i * 64
        for u in range(4):
            sl = pl.ds(base + u * 16, 16)
            mine = table[sl]
            other = table2[sl]
            hi = jnp.where(half == 1, mine, other)
            lo = jnp.where(half == 1, other, mine)
            table[sl] = jnp.where(hi != SENT, hi, lo)
        return carry
    lax.fori_loop(0, M // 64, merge, 0)

    pend_o = [None, None]
    for g in range(GQ):
        slot = g % 2
        pend_q[g].wait()
        if pend_o[slot] is not None:
            pend_o[slot].wait()

        def query_body(i, c2, _slot=slot):
            base = i * 64
            for u in range(4):
                sl = pl.ds(base + u * 16, 16)
                w = qbuf[_slot, sl]
                alo = w & 0xFFFF
                ahi = lax.shift_right_logical(w, 16)
                obuf[_slot, pl.ds(base + u * 16, 16)] = (
                    plsc.load_gather(table, [alo]))
                obuf[_slot, pl.ds(CH // 2 + base + u * 16, 16)] = (
                    plsc.load_gather(table, [ahi]))
            return c2
        lax.fori_loop(0, CH // 128, query_body, 0)

        pend_o[slot] = pltpu.async_copy(
            obuf.at[slot], outT_hbm.at[gbase + g, nn], semo[slot])
        if g + 2 < GQ:
            pend_q.append(query_start(g + 2))
    pend_o[0].wait()
    pend_o[1].wait()


def _sc_run(memory, addrT, tgtT, qaddrT):
    mesh = plsc.VectorSubcoreMesh(core_axis_name="c", subcore_axis_name="s")
    run = pl.kernel(
        _sc_body,
        out_type=jax.ShapeDtypeStruct((G, N, CH), jnp.float32),
        mesh=mesh,
        scratch_types=[
            pltpu.VMEM((M,), jnp.float32),
            pltpu.VMEM((M,), jnp.float32),
            pltpu.VMEM((2, CH // 2), jnp.int32),
            pltpu.VMEM((2, CH), jnp.float32),
            pltpu.VMEM((2, CH // 2), jnp.int32),
            pltpu.VMEM((2, CH), jnp.float32),
            pltpu.VMEM_SHARED((NSUB, M), jnp.float32),
            pltpu.SemaphoreType.DMA,
            pltpu.SemaphoreType.DMA,
            pltpu.SemaphoreType.DMA,
            pltpu.SemaphoreType.DMA,
            pltpu.SemaphoreType.DMA,
            pltpu.SemaphoreType.DMA,
        ],
        compiler_params=pltpu.CompilerParams(needs_layout_passes=False),
    )
    return run(memory, addrT, tgtT, qaddrT)


def _post_body(outT_ref, out_ref):
    out_ref[...] = outT_ref[0].T


def _post(outT):
    return pl.pallas_call(
        _post_body,
        grid=(G,),
        in_specs=[pl.BlockSpec((1, N, CH), lambda i: (i, 0, 0))],
        out_specs=pl.BlockSpec((CH, N), lambda i: (i, 0)),
        out_shape=jax.ShapeDtypeStruct((Q, N), jnp.float32),
    )(outT)


def kernel(memory, target_bits, ctx_bits, query_bits, mapping):
    addrT, qaddrT, tgtT = _prep(ctx_bits, query_bits, target_bits, mapping)
    outT = _sc_run(memory, addrT, tgtT, qaddrT)
    return _post(outT)


# speedup vs baseline: 96.2763x; 1.1073x over previous
import jax
import jax.numpy as jnp
from jax import lax
from jax.experimental import pallas as pl
from jax.experimental.pallas import tpu as pltpu
from jax.experimental.pallas import tpu_sc as plsc

B = 262144
Q = 262144
TI = 32
N = 16
NB = 14
M = 2 ** NB

NCORE = 2
NSUB = 16
BH = B // 2
QH = Q // 2
CH = 4096
G = B // CH
SENT = -(2.0 ** 40)


def _prep_body(mapping_ref, ctx_ref, qry_ref, tgt_ref,
               addrT_ref, qaddrT_ref, tgtT_ref):
    mp = mapping_ref[...]
    i3 = lax.broadcasted_iota(jnp.int32, (TI, 2 * N, NB), 0)
    n3 = lax.broadcasted_iota(jnp.int32, (TI, 2 * N, NB), 1)
    k3 = lax.broadcasted_iota(jnp.int32, (TI, 2 * N, NB), 2)
    sel_lo = (n3 < N) & (k3 < 7)
    valid = sel_lo | ((n3 >= N) & (k3 >= 7))
    shift = jnp.where(valid, jnp.where(sel_lo, k3, k3 - 7), 0)
    p3 = jnp.where(valid, jnp.int32(1) << shift, 0)
    mp2 = jnp.concatenate([mp, mp], axis=0)
    w2 = jnp.sum(jnp.where(mp2[None] == i3, p3, 0), axis=2)
    wb = w2.astype(jnp.bfloat16)

    def addresses(bits_i32):
        bits = bits_i32.astype(jnp.bfloat16)
        r = jax.lax.dot(bits, wb,
                        preferred_element_type=jnp.float32)
        rt = r.T
        a = rt[:N] + 128.0 * rt[N:]
        ai = a.astype(jnp.int32)
        return ai[:, :CH // 2] | (ai[:, CH // 2:] << 16)

    addrT_ref[...] = addresses(ctx_ref[...])[None]
    qaddrT_ref[...] = addresses(qry_ref[...])[None]
    tgtT_ref[...] = tgt_ref[...].T[None]


def _prep(ctx_bits, query_bits, target_bits, mapping):
    return pl.pallas_call(
        _prep_body,
        grid=(G,),
        in_specs=[
            pl.BlockSpec((N, NB), lambda i: (0, 0)),
            pl.BlockSpec((CH, TI), lambda i: (i, 0)),
            pl.BlockSpec((CH, TI), lambda i: (i, 0)),
            pl.BlockSpec((CH, N), lambda i: (i, 0)),
        ],
        out_specs=[
            pl.BlockSpec((1, N, CH // 2), lambda i: (i, 0, 0)),
            pl.BlockSpec((1, N, CH // 2), lambda i: (i, 0, 0)),
            pl.BlockSpec((1, N, CH), lambda i: (i, 0, 0)),
        ],
        out_shape=[
            jax.ShapeDtypeStruct((G, N, CH // 2), jnp.int32),
            jax.ShapeDtypeStruct((G, N, CH // 2), jnp.int32),
            jax.ShapeDtypeStruct((G, N, CH), jnp.float32),
        ],
    )(mapping, ctx_bits, query_bits, target_bits)


def _sc_body(mem_hbm, addrT_hbm, tgtT_hbm, qaddrT_hbm, outT_hbm,
             table, table2, abuf, tbuf, qbuf, obuf, shared,
             semc0, semc1, semq0, semq1, semo0, semo1):
    c = lax.axis_index("c")
    s = lax.axis_index("s")
    nn_loc = s % (N // NCORE)
    half = s // (N // NCORE)
    nn = c * (N // NCORE) + nn_loc

    semc = [semc0, semc1]
    semq = [semq0, semq1]
    semo = [semo0, semo1]

    gbase = half * (G // 2)
    GC = G // 2

    def commit_start(g):
        slot = g % 2
        ha = pltpu.async_copy(addrT_hbm.at[gbase + g, nn],
                              abuf.at[slot], semc[slot])
        hv = pltpu.async_copy(tgtT_hbm.at[gbase + g, nn],
                              tbuf.at[slot], semc[slot])
        return (ha, hv)

    pend_c = [commit_start(0), commit_start(1)]

    @pl.when(half == 0)
    def _():
        pltpu.sync_copy(mem_hbm.at[nn], table)

    @pl.when(half == 1)
    def _():
        def fill(i, carry):
            table[pl.ds(i * 16, 16)] = jnp.full((16,), SENT, jnp.float32)
            return carry
        lax.fori_loop(0, M // 16, fill, 0)

    for g in range(GC):
        slot = g % 2
        ha, hv = pend_c[g]
        ha.wait()
        hv.wait()

        def commit_body(i, c2, _slot=slot, _half=0):
            base = i * 64
            for u in range(4):
                sl = pl.ds(base + u * 16, 16)
                w = abuf[_slot, sl]
                a = lax.shift_right_logical(w, 16) if _half else (w & 0xFFFF)
                v = tbuf[_slot, pl.ds(_half * (CH // 2) + base + u * 16, 16)]
                plsc.store_scatter(table, [a], v)
            return c2
        lax.fori_loop(0, CH // 128, commit_body, 0)
        lax.fori_loop(0, CH // 128,
                      lambda i, c2, _s=slot: commit_body(i, c2, _s, 1), 0)
        if g + 2 < GC:
            pend_c.append(commit_start(g + 2))

    GQ = G // 2

    def query_start(g):
        slot = g % 2
        return pltpu.async_copy(qaddrT_hbm.at[gbase + g, nn],
                                qbuf.at[slot], semq[slot])

    pend_q = [query_start(0), query_start(1)]

    plsc.subcore_barrier()
    pltpu.sync_copy(table, shared.at[s])
    plsc.subcore_barrier()
    sib = jnp.where(half == 0, s + N // NCORE, s - N // NCORE)
    pltpu.sync_copy(shared.at[sib], table2)

    def merge(i, carry):
        base = i * 64
        for u in range(4):
            sl = pl.ds(base + u * 16, 16)
            mine = table[sl]
            other = table2[sl]
            hi = jnp.where(half == 1, mine, other)
            lo = jnp.where(half == 1, other, mine)
            table[sl] = jnp.where(hi != SENT, hi, lo)
        return carry
    lax.fori_loop(0, M // 64, merge, 0)

    pend_o = [None, None]
    for g in range(GQ):
        slot = g % 2
        pend_q[g].wait()
        if pend_o[slot] is not None:
            pend_o[slot].wait()

        def query_body(i, c2, _slot=slot):
            base = i * 64
            for u in range(4):
                sl = pl.ds(base + u * 16, 16)
                w = qbuf[_slot, sl]
                alo = w & 0xFFFF
                ahi = lax.shift_right_logical(w, 16)
                obuf[_slot, pl.ds(base + u * 16, 16)] = (
                    plsc.load_gather(table, [alo]))
                obuf[_slot, pl.ds(CH // 2 + base + u * 16, 16)] = (
                    plsc.load_gather(table, [ahi]))
            return c2
        lax.fori_loop(0, CH // 128, query_body, 0)

        pend_o[slot] = pltpu.async_copy(
            obuf.at[slot], outT_hbm.at[gbase + g, nn], semo[slot])
        if g + 2 < GQ:
            pend_q.append(query_start(g + 2))
    pend_o[0].wait()
    pend_o[1].wait()


def _sc_run(memory, addrT, tgtT, qaddrT):
    mesh = plsc.VectorSubcoreMesh(core_axis_name="c", subcore_axis_name="s")
    run = pl.kernel(
        _sc_body,
        out_type=jax.ShapeDtypeStruct((G, N, CH), jnp.float32),
        mesh=mesh,
        scratch_types=[
            pltpu.VMEM((M,), jnp.float32),
            pltpu.VMEM((M,), jnp.float32),
            pltpu.VMEM((2, CH // 2), jnp.int32),
            pltpu.VMEM((2, CH), jnp.float32),
            pltpu.VMEM((2, CH // 2), jnp.int32),
            pltpu.VMEM((2, CH), jnp.float32),
            pltpu.VMEM_SHARED((NSUB, M), jnp.float32),
            pltpu.SemaphoreType.DMA,
            pltpu.SemaphoreType.DMA,
            pltpu.SemaphoreType.DMA,
            pltpu.SemaphoreType.DMA,
            pltpu.SemaphoreType.DMA,
            pltpu.SemaphoreType.DMA,
        ],
        compiler_params=pltpu.CompilerParams(needs_layout_passes=False),
    )
    return run(memory, addrT, tgtT, qaddrT)


def _post_body(outT_ref, out_ref):
    out_ref[...] = outT_ref[0].T


def _post(outT):
    return pl.pallas_call(
        _post_body,
        grid=(G,),
        in_specs=[pl.BlockSpec((1, N, CH), lambda i: (i, 0, 0))],
        out_specs=pl.BlockSpec((CH, N), lambda i: (i, 0)),
        out_shape=jax.ShapeDtypeStruct((Q, N), jnp.float32),
    )(outT)


def kernel(memory, target_bits, ctx_bits, query_bits, mapping):
    addrT, qaddrT, tgtT = _prep(ctx_bits, query_bits, target_bits, mapping)
    outT = _sc_run(memory, addrT, tgtT, qaddrT)
    return _post(outT)
